# initial kernel scaffold (unmeasured)
import jax
import jax.numpy as jnp
from jax import lax
from jax.experimental import pallas as pl
from jax.experimental.pallas import tpu as pltpu

N_DEV = 4


def kernel(Q, K, V):
    b, s_loc, h, d = Q.shape
    scale = d ** -0.5

    def body(q_ref, k_ref, v_ref, out_ref, kbuf, vbuf, send_sems, recv_sems):
        my = lax.axis_index("i")
        left = (my - 1) % N_DEV
        right = (my + 1) % N_DEV

        barrier = pltpu.get_barrier_semaphore()
        for nbr in (left, right):
            pl.semaphore_signal(
                barrier, inc=1,
                device_id=(nbr,), device_id_type=pl.DeviceIdType.MESH,
            )
        pl.semaphore_wait(barrier, 2)

        kbuf[0] = k_ref[...].astype(jnp.bfloat16)
        vbuf[0] = v_ref[...].astype(jnp.bfloat16)

        for t in range(N_DEV - 1):
            krdma = pltpu.make_async_remote_copy(
                src_ref=kbuf.at[t],
                dst_ref=kbuf.at[t + 1],
                send_sem=send_sems.at[2 * t],
                recv_sem=recv_sems.at[2 * t],
                device_id=(right,),
                device_id_type=pl.DeviceIdType.MESH,
            )
            vrdma = pltpu.make_async_remote_copy(
                src_ref=vbuf.at[t],
                dst_ref=vbuf.at[t + 1],
                send_sem=send_sems.at[2 * t + 1],
                recv_sem=recv_sems.at[2 * t + 1],
                device_id=(right,),
                device_id_type=pl.DeviceIdType.MESH,
            )
            krdma.start()
            vrdma.start()
            krdma.wait()
            vrdma.wait()

        for bb in range(b):
            for hh in range(h):
                q = q_ref[bb, :, hh, :].astype(jnp.bfloat16)
                s_parts = []
                for j in range(N_DEV):
                    kj = kbuf[j, bb, :, hh, :]
                    s = lax.dot_general(
                        q, kj,
                        (((1,), (1,)), ((), ())),
                        preferred_element_type=jnp.float32,
                    )
                    s_parts.append(s * scale)
                m = s_parts[0].max(axis=1, keepdims=True)
                for j in range(1, N_DEV):
                    m = jnp.maximum(m, s_parts[j].max(axis=1, keepdims=True))
                l = jnp.zeros((s_loc, 1), jnp.float32)
                acc = jnp.zeros((s_loc, d), jnp.float32)
                for j in range(N_DEV):
                    p = jnp.exp(s_parts[j] - m)
                    l = l + p.sum(axis=1, keepdims=True)
                    vj = vbuf[j, bb, :, hh, :]
                    acc = acc + lax.dot_general(
                        p.astype(jnp.bfloat16), vj,
                        (((1,), (0,)), ((), ())),
                        preferred_element_type=jnp.float32,
                    )
                out_ref[bb, :, hh, :] = acc / l

    return pl.pallas_call(
        body,
        out_shape=jax.ShapeDtypeStruct((b, s_loc, h, d), jnp.float32),
        in_specs=[
            pl.BlockSpec(memory_space=pltpu.VMEM),
            pl.BlockSpec(memory_space=pltpu.VMEM),
            pl.BlockSpec(memory_space=pltpu.VMEM),
        ],
        out_specs=pl.BlockSpec(memory_space=pltpu.VMEM),
        scratch_shapes=[
            pltpu.VMEM((N_DEV, b, s_loc, h, d), jnp.bfloat16),
            pltpu.VMEM((N_DEV, b, s_loc, h, d), jnp.bfloat16),
            pltpu.SemaphoreType.DMA((2 * (N_DEV - 1),)),
            pltpu.SemaphoreType.DMA((2 * (N_DEV - 1),)),
        ],
        compiler_params=pltpu.CompilerParams(collective_id=0),
    )(Q, K, V)


# baseline (device time: 209733 ns/iter reference)
import jax
import jax.numpy as jnp
from jax import lax
from jax.experimental import pallas as pl
from jax.experimental.pallas import tpu as pltpu

N_DEV = 4


def kernel(Q, K, V):
    b, s_loc, h, d = Q.shape
    scale = d ** -0.5

    def body(q_ref, k_ref, v_ref, out_ref,
             qbuf, kbuf, vbuf, obuf, send_sems, recv_sems):
        my = lax.axis_index("i")
        left = (my - 1) % N_DEV
        right = (my + 1) % N_DEV

        barrier = pltpu.get_barrier_semaphore()
        for nbr in (left, right):
            pl.semaphore_signal(
                barrier, inc=1,
                device_id=(nbr,), device_id_type=pl.DeviceIdType.MESH,
            )
        pl.semaphore_wait(barrier, 2)

        for bb in range(b):
            for hh in range(h):
                qbuf[bb, hh] = q_ref[bb, :, hh, :].astype(jnp.bfloat16)
                kbuf[0, bb, hh] = k_ref[bb, :, hh, :].astype(jnp.bfloat16)
                vbuf[0, bb, hh] = v_ref[bb, :, hh, :].astype(jnp.bfloat16)

        for t in range(N_DEV - 1):
            krdma = pltpu.make_async_remote_copy(
                src_ref=kbuf.at[t],
                dst_ref=kbuf.at[t + 1],
                send_sem=send_sems.at[2 * t],
                recv_sem=recv_sems.at[2 * t],
                device_id=(right,),
                device_id_type=pl.DeviceIdType.MESH,
            )
            vrdma = pltpu.make_async_remote_copy(
                src_ref=vbuf.at[t],
                dst_ref=vbuf.at[t + 1],
                send_sem=send_sems.at[2 * t + 1],
                recv_sem=recv_sems.at[2 * t + 1],
                device_id=(right,),
                device_id_type=pl.DeviceIdType.MESH,
            )
            krdma.start()
            vrdma.start()
            krdma.wait()
            vrdma.wait()

        def compute_bh(idx, carry):
            bb = idx // h
            hh = idx % h
            q = qbuf[bb, hh]
            m = jnp.full((s_loc, 1), -jnp.inf, jnp.float32)
            l = jnp.zeros((s_loc, 1), jnp.float32)
            acc = jnp.zeros((s_loc, d), jnp.float32)
            for j in range(N_DEV):
                kj = kbuf[j, bb, hh]
                s = lax.dot_general(
                    q, kj,
                    (((1,), (1,)), ((), ())),
                    preferred_element_type=jnp.float32,
                ) * scale
                m_new = jnp.maximum(m, s.max(axis=1, keepdims=True))
                alpha = jnp.exp(m - m_new)
                p = jnp.exp(s - m_new)
                l = l * alpha + p.sum(axis=1, keepdims=True)
                vj = vbuf[j, bb, hh]
                acc = acc * alpha + lax.dot_general(
                    p.astype(jnp.bfloat16), vj,
                    (((1,), (0,)), ((), ())),
                    preferred_element_type=jnp.float32,
                )
                m = m_new
            obuf[bb, hh] = acc / l
            return carry

        lax.fori_loop(0, b * h, compute_bh, 0)

        for bb in range(b):
            for hh in range(h):
                out_ref[bb, :, hh, :] = obuf[bb, hh]

    return pl.pallas_call(
        body,
        out_shape=jax.ShapeDtypeStruct((b, s_loc, h, d), jnp.float32),
        in_specs=[
            pl.BlockSpec(memory_space=pltpu.VMEM),
            pl.BlockSpec(memory_space=pltpu.VMEM),
            pl.BlockSpec(memory_space=pltpu.VMEM),
        ],
        out_specs=pl.BlockSpec(memory_space=pltpu.VMEM),
        scratch_shapes=[
            pltpu.VMEM((b, h, s_loc, d), jnp.bfloat16),
            pltpu.VMEM((N_DEV, b, h, s_loc, d), jnp.bfloat16),
            pltpu.VMEM((N_DEV, b, h, s_loc, d), jnp.bfloat16),
            pltpu.VMEM((b, h, s_loc, d), jnp.float32),
            pltpu.SemaphoreType.DMA((2 * (N_DEV - 1),)),
            pltpu.SemaphoreType.DMA((2 * (N_DEV - 1),)),
        ],
        compiler_params=pltpu.CompilerParams(
            collective_id=0,
            vmem_limit_bytes=100 * 1024 * 1024,
        ),
    )(Q, K, V)


# device time: 76531 ns/iter; 2.7405x vs baseline; 2.7405x over previous
import functools
import os

import jax
import jax.numpy as jnp
from jax import lax
from jax.experimental import pallas as pl
from jax.experimental.pallas import tpu as pltpu

N_DEV = 4

_SKIP_RING = bool(int(os.environ.get("KERNEL_SKIP_RING", "0")))
_SKIP_COMPUTE = bool(int(os.environ.get("KERNEL_SKIP_COMPUTE", "0")))


def kernel(Q, K, V):
    b, s_loc, h, d = Q.shape
    scale = d ** -0.5

    Qp = (Q * scale).astype(jnp.bfloat16).transpose(0, 2, 1, 3)
    Kp = K.astype(jnp.bfloat16).transpose(0, 2, 3, 1)
    Vp = V.astype(jnp.bfloat16).transpose(0, 2, 3, 1)

    def body(q_ref, k_ref, v_ref, out_ref,
             kbuf, vbuf, accbuf, lbuf, send_sems, recv_sems):
        my = lax.axis_index("i")
        left = (my - 1) % N_DEV
        right = (my + 1) % N_DEV

        barrier = pltpu.get_barrier_semaphore()
        for nbr in (left, right):
            pl.semaphore_signal(
                barrier, inc=1,
                device_id=(nbr,), device_id_type=pl.DeviceIdType.MESH,
            )
        pl.semaphore_wait(barrier, 2)

        def remote(src, dst, sem_idx, target):
            return pltpu.make_async_remote_copy(
                src_ref=src, dst_ref=dst,
                send_sem=send_sems.at[sem_idx],
                recv_sem=recv_sems.at[sem_idx],
                device_id=(target,),
                device_id_type=pl.DeviceIdType.MESH,
            )

        k_r = remote(k_ref, kbuf.at[0], 0, right)
        v_r = remote(v_ref, vbuf.at[0], 1, right)
        k_l = remote(k_ref, kbuf.at[1], 2, left)
        v_l = remote(v_ref, vbuf.at[1], 3, left)
        k_fwd = remote(kbuf.at[0], kbuf.at[2], 4, right)
        v_fwd = remote(vbuf.at[1], vbuf.at[2], 5, left)

        if not _SKIP_RING:
            k_r.start()
            v_r.start()
            k_l.start()
            v_l.start()

        def phase(kk, vv, first=False, last=False):
            def step(idx, c):
                bb = idx // h
                hh = idx % h
                q = q_ref[bb, hh]
                kT = kk[bb, hh]
                s = lax.dot_general(
                    q, kT,
                    (((1,), (0,)), ((), ())),
                    preferred_element_type=jnp.float32,
                )
                p = jnp.exp(s)
                lsum = p.sum(axis=1, keepdims=True)
                vT = vv[bb, hh]
                pv = lax.dot_general(
                    p.astype(jnp.bfloat16), vT,
                    (((1,), (1,)), ((), ())),
                    preferred_element_type=jnp.float32,
                )
                if first:
                    accbuf[bb, hh] = pv
                    lbuf[bb, hh] = lsum
                elif last:
                    out_ref[bb, hh] = (accbuf[bb, hh] + pv) / (lbuf[bb, hh] + lsum)
                else:
                    accbuf[bb, hh] = accbuf[bb, hh] + pv
                    lbuf[bb, hh] = lbuf[bb, hh] + lsum
                return c

            if not _SKIP_COMPUTE:
                lax.fori_loop(0, b * h, step, 0)

        phase(k_ref, v_ref, first=True)

        if not _SKIP_RING:
            k_r.wait_recv()
            v_r.wait_recv()
            k_fwd.start()
        phase(kbuf.at[0], vbuf.at[0])

        if not _SKIP_RING:
            k_l.wait_recv()
            v_l.wait_recv()
            v_fwd.start()
        phase(kbuf.at[1], vbuf.at[1])

        if not _SKIP_RING:
            k_fwd.wait_recv()
            v_fwd.wait_recv()
        phase(kbuf.at[2], vbuf.at[2], last=True)

        if not _SKIP_RING:
            for r in (k_r, v_r, k_l, v_l, k_fwd, v_fwd):
                r.wait_send()

        @functools.partial(
            pl.run_scoped, second_barrier=pltpu.SemaphoreType.REGULAR
        )
        def _(second_barrier):
            for nbr in (left, right):
                pl.semaphore_signal(
                    second_barrier, inc=1,
                    device_id=(nbr,), device_id_type=pl.DeviceIdType.MESH,
                )
            pl.semaphore_wait(second_barrier, 2)

    out_p = pl.pallas_call(
        body,
        out_shape=jax.ShapeDtypeStruct((b, h, s_loc, d), jnp.float32),
        in_specs=[
            pl.BlockSpec(memory_space=pltpu.VMEM),
            pl.BlockSpec(memory_space=pltpu.VMEM),
            pl.BlockSpec(memory_space=pltpu.VMEM),
        ],
        out_specs=pl.BlockSpec(memory_space=pltpu.VMEM),
        scratch_shapes=[
            pltpu.VMEM((3, b, h, d, s_loc), jnp.bfloat16),
            pltpu.VMEM((3, b, h, d, s_loc), jnp.bfloat16),
            pltpu.VMEM((b, h, s_loc, d), jnp.float32),
            pltpu.VMEM((b, h, s_loc, 1), jnp.float32),
            pltpu.SemaphoreType.DMA((6,)),
            pltpu.SemaphoreType.DMA((6,)),
        ],
        compiler_params=pltpu.CompilerParams(
            collective_id=0,
            vmem_limit_bytes=100 * 1024 * 1024,
        ),
    )(Qp, Kp, Vp)

    return out_p.transpose(0, 2, 1, 3)


# device time: 69543 ns/iter; 3.0159x vs baseline; 1.1005x over previous
import functools
import os

import jax
import jax.numpy as jnp
from jax import lax
from jax.experimental import pallas as pl
from jax.experimental.pallas import tpu as pltpu

N_DEV = 4

_SKIP_RING = bool(int(os.environ.get("KERNEL_SKIP_RING", "0")))
_SKIP_COMPUTE = bool(int(os.environ.get("KERNEL_SKIP_COMPUTE", "0")))


def kernel(Q, K, V):
    b, s_loc, h, d = Q.shape
    h2 = h // 2
    d2 = 2 * d
    scale = d ** -0.5

    Qp = (Q * scale).astype(jnp.bfloat16).reshape(b, s_loc, h2, d2)
    Qp = Qp.transpose(0, 2, 1, 3)
    Kp = K.astype(jnp.bfloat16).reshape(b, s_loc, h2, d2)
    Kp = Kp.transpose(0, 2, 3, 1)
    Vp = V.astype(jnp.bfloat16).reshape(b, s_loc, h2, d2)
    Vp = Vp.transpose(0, 2, 1, 3)

    def body(q_ref, k_ref, v_ref, out_ref,
             kbuf, vbuf, accbuf, lbuf, send_sems, recv_sems):
        my = lax.axis_index("i")
        left = (my - 1) % N_DEV
        right = (my + 1) % N_DEV

        barrier = pltpu.get_barrier_semaphore()
        for nbr in (left, right):
            pl.semaphore_signal(
                barrier, inc=1,
                device_id=(nbr,), device_id_type=pl.DeviceIdType.MESH,
            )
        pl.semaphore_wait(barrier, 2)

        def remote(src, dst, sem_idx, target):
            return pltpu.make_async_remote_copy(
                src_ref=src, dst_ref=dst,
                send_sem=send_sems.at[sem_idx],
                recv_sem=recv_sems.at[sem_idx],
                device_id=(target,),
                device_id_type=pl.DeviceIdType.MESH,
            )

        k_r = remote(k_ref, kbuf.at[0], 0, right)
        v_r = remote(v_ref, vbuf.at[0], 1, right)
        k_l = remote(k_ref, kbuf.at[1], 2, left)
        v_l = remote(v_ref, vbuf.at[1], 3, left)
        k_fwd = remote(kbuf.at[0], kbuf.at[2], 4, right)
        v_fwd = remote(vbuf.at[1], vbuf.at[2], 5, left)

        if not _SKIP_RING:
            k_r.start()
            v_r.start()
            k_l.start()
            v_l.start()

        def phase(kk, vv, first=False, last=False):
            def step(idx, c):
                bb = idx // h2
                hp = idx % h2
                qq = q_ref[bb, hp]
                kk2 = kk[bb, hp]
                vv2 = vv[bb, hp]
                pvs = []
                lsums = []
                for t in range(2):
                    st = lax.dot_general(
                        qq[:, t * d:(t + 1) * d],
                        kk2[t * d:(t + 1) * d, :],
                        (((1,), (0,)), ((), ())),
                        preferred_element_type=jnp.float32,
                    )
                    p = jnp.exp(st)
                    lsums.append(p.sum(axis=1, keepdims=True))
                    pvs.append(lax.dot_general(
                        p.astype(jnp.bfloat16),
                        vv2[:, t * d:(t + 1) * d],
                        (((1,), (0,)), ((), ())),
                        preferred_element_type=jnp.float32,
                    ))
                pv = jnp.concatenate(pvs, axis=1)
                lsum = jnp.concatenate(
                    [jnp.broadcast_to(ls, (s_loc, d)) for ls in lsums],
                    axis=1,
                )
                if first:
                    accbuf[bb, hp] = pv
                    lbuf[bb, hp] = lsum
                elif last:
                    out_ref[bb, hp] = (accbuf[bb, hp] + pv) / (lbuf[bb, hp] + lsum)
                else:
                    accbuf[bb, hp] = accbuf[bb, hp] + pv
                    lbuf[bb, hp] = lbuf[bb, hp] + lsum
                return c

            if not _SKIP_COMPUTE:
                lax.fori_loop(0, b * h2, step, 0)

        phase(k_ref, v_ref, first=True)

        if not _SKIP_RING:
            k_r.wait_recv()
            k_fwd.start()
            v_r.wait_recv()
        phase(kbuf.at[0], vbuf.at[0])

        if not _SKIP_RING:
            v_l.wait_recv()
            v_fwd.start()
            k_l.wait_recv()
        phase(kbuf.at[1], vbuf.at[1])

        if not _SKIP_RING:
            k_fwd.wait_recv()
            v_fwd.wait_recv()
        phase(kbuf.at[2], vbuf.at[2], last=True)

        if not _SKIP_RING:
            for r in (k_r, v_r, k_l, v_l, k_fwd, v_fwd):
                r.wait_send()

        @functools.partial(
            pl.run_scoped, second_barrier=pltpu.SemaphoreType.REGULAR
        )
        def _(second_barrier):
            for nbr in (left, right):
                pl.semaphore_signal(
                    second_barrier, inc=1,
                    device_id=(nbr,), device_id_type=pl.DeviceIdType.MESH,
                )
            pl.semaphore_wait(second_barrier, 2)

    out_p = pl.pallas_call(
        body,
        out_shape=jax.ShapeDtypeStruct((b, h2, s_loc, d2), jnp.float32),
        in_specs=[
            pl.BlockSpec(memory_space=pltpu.VMEM),
            pl.BlockSpec(memory_space=pltpu.VMEM),
            pl.BlockSpec(memory_space=pltpu.VMEM),
        ],
        out_specs=pl.BlockSpec(memory_space=pltpu.VMEM),
        scratch_shapes=[
            pltpu.VMEM((3, b, h2, d2, s_loc), jnp.bfloat16),
            pltpu.VMEM((3, b, h2, s_loc, d2), jnp.bfloat16),
            pltpu.VMEM((b, h2, s_loc, d2), jnp.float32),
            pltpu.VMEM((b, h2, s_loc, d2), jnp.float32),
            pltpu.SemaphoreType.DMA((6,)),
            pltpu.SemaphoreType.DMA((6,)),
        ],
        compiler_params=pltpu.CompilerParams(
            collective_id=0,
            vmem_limit_bytes=100 * 1024 * 1024,
        ),
    )(Qp, Kp, Vp)

    return out_p.transpose(0, 2, 1, 3).reshape(b, s_loc, h, d)


# device time: 64833 ns/iter; 3.2350x vs baseline; 1.0726x over previous
import functools
import os

import jax
import jax.numpy as jnp
from jax import lax
from jax.experimental import pallas as pl
from jax.experimental.pallas import tpu as pltpu

N_DEV = 4

_SKIP_RING = bool(int(os.environ.get("KERNEL_SKIP_RING", "0")))
_SKIP_COMPUTE = bool(int(os.environ.get("KERNEL_SKIP_COMPUTE", "0")))
_RING_MODE = os.environ.get("KERNEL_RING_MODE", "full")
_COMPUTE_MODE = int(os.environ.get("KERNEL_COMPUTE_MODE", "0"))


def kernel(Q, K, V):
    b, s_loc, h, d = Q.shape
    h2 = h // 2
    d2 = 2 * d
    scale = d ** -0.5

    def prep(x):
        return x.reshape(b, s_loc, h2, d2).transpose(0, 2, 1, 3)

    Qp = prep((Q * scale).astype(jnp.bfloat16))
    Kp = prep(K.astype(jnp.bfloat16))
    Vp = prep(V.astype(jnp.bfloat16))

    def body(q_ref, k_ref, v_ref, out_ref,
             kbuf, vbuf, accbuf, lbuf, send_sems, recv_sems):
        my = lax.axis_index("i")
        left = (my - 1) % N_DEV
        right = (my + 1) % N_DEV

        barrier = pltpu.get_barrier_semaphore()
        for nbr in (left, right):
            pl.semaphore_signal(
                barrier, inc=1,
                device_id=(nbr,), device_id_type=pl.DeviceIdType.MESH,
            )
        pl.semaphore_wait(barrier, 2)

        def remote(src, dst, sem_idx, target):
            return pltpu.make_async_remote_copy(
                src_ref=src, dst_ref=dst,
                send_sem=send_sems.at[sem_idx],
                recv_sem=recv_sems.at[sem_idx],
                device_id=(target,),
                device_id_type=pl.DeviceIdType.MESH,
            )

        k_r = remote(k_ref, kbuf.at[0], 0, right)
        v_r = remote(v_ref, vbuf.at[0], 1, right)
        k_l = remote(k_ref, kbuf.at[1], 2, left)
        v_l = remote(v_ref, vbuf.at[1], 3, left)
        k_fwd = remote(kbuf.at[0], kbuf.at[2], 4, right)
        v_fwd = remote(vbuf.at[1], vbuf.at[2], 5, left)

        _do_v = _RING_MODE in ("full", "hop0")
        _do_fwd = _RING_MODE == "full"
        if not _SKIP_RING:
            k_r.start()
            if _do_v:
                v_r.start()
            k_l.start()
            if _do_v:
                v_l.start()

        def head_chunk(q_t, k_t, v_t):
            st = lax.dot_general(
                q_t, k_t,
                (((1,), (1,)), ((), ())),
                preferred_element_type=jnp.float32,
            )
            p = st if _COMPUTE_MODE in (1, 3) else jnp.exp(st)
            if _COMPUTE_MODE in (2, 3):
                lsum = jnp.ones((s_loc, 1), jnp.float32)
            else:
                lsum = p.sum(axis=1, keepdims=True)
            pv = lax.dot_general(
                p.astype(jnp.bfloat16), v_t,
                (((1,), (0,)), ((), ())),
                preferred_element_type=jnp.float32,
            )
            return pv, lsum

        def make_phase(slots, mode):
            def step(idx, c):
                bb = idx // h2
                hp = idx % h2
                qq = q_ref[bb, hp]
                pvs = [jnp.zeros((s_loc, d), jnp.float32) for _ in range(2)]
                lsums = [jnp.zeros((s_loc, 1), jnp.float32) for _ in range(2)]
                for j in slots:
                    kk2 = (k_ref if j is None else kbuf.at[j])[bb, hp]
                    vv2 = (v_ref if j is None else vbuf.at[j])[bb, hp]
                    for t in range(2):
                        pv, ls = head_chunk(
                            qq[:, t * d:(t + 1) * d],
                            kk2[:, t * d:(t + 1) * d],
                            vv2[:, t * d:(t + 1) * d],
                        )
                        pvs[t] = pvs[t] + pv
                        lsums[t] = lsums[t] + ls
                pv = jnp.concatenate(pvs, axis=1)
                if mode == "first":
                    accbuf[bb, hp] = pv
                    for t in range(2):
                        lbuf[bb, 2 * hp + t] = lsums[t]
                elif mode == "mid":
                    accbuf[bb, hp] = accbuf[bb, hp] + pv
                    for t in range(2):
                        lbuf[bb, 2 * hp + t] = lbuf[bb, 2 * hp + t] + lsums[t]
                else:
                    tot = accbuf[bb, hp] + pv
                    outs = [
                        tot[:, t * d:(t + 1) * d]
                        / (lbuf[bb, 2 * hp + t] + lsums[t])
                        for t in range(2)
                    ]
                    out_ref[bb, hp] = jnp.concatenate(
                        outs, axis=1).astype(jnp.bfloat16)
                return c

            if not _SKIP_COMPUTE:
                lax.fori_loop(0, b * h2, step, 0)

        make_phase([None], "first")

        if not _SKIP_RING:
            k_r.wait_recv()
            if _do_fwd:
                k_fwd.start()
            k_l.wait_recv()
            if _do_v:
                v_l.wait_recv()
                if _do_fwd:
                    v_fwd.start()
                v_r.wait_recv()
        make_phase([0, 1], "mid")

        if not _SKIP_RING and _do_fwd:
            k_fwd.wait_recv()
            v_fwd.wait_recv()
        make_phase([2], "last")

        if not _SKIP_RING:
            drain = [k_r, k_l]
            if _do_v:
                drain += [v_r, v_l]
            if _do_fwd:
                drain += [k_fwd, v_fwd]
            for r in drain:
                r.wait_send()

        @functools.partial(
            pl.run_scoped, second_barrier=pltpu.SemaphoreType.REGULAR
        )
        def _(second_barrier):
            for nbr in (left, right):
                pl.semaphore_signal(
                    second_barrier, inc=1,
                    device_id=(nbr,), device_id_type=pl.DeviceIdType.MESH,
                )
            pl.semaphore_wait(second_barrier, 2)

    out_p = pl.pallas_call(
        body,
        out_shape=jax.ShapeDtypeStruct((b, h2, s_loc, d2), jnp.bfloat16),
        in_specs=[
            pl.BlockSpec(memory_space=pltpu.VMEM),
            pl.BlockSpec(memory_space=pltpu.VMEM),
            pl.BlockSpec(memory_space=pltpu.VMEM),
        ],
        out_specs=pl.BlockSpec(memory_space=pltpu.VMEM),
        scratch_shapes=[
            pltpu.VMEM((3, b, h2, s_loc, d2), jnp.bfloat16),
            pltpu.VMEM((3, b, h2, s_loc, d2), jnp.bfloat16),
            pltpu.VMEM((b, h2, s_loc, d2), jnp.float32),
            pltpu.VMEM((b, h, s_loc, 1), jnp.float32),
            pltpu.SemaphoreType.DMA((6,)),
            pltpu.SemaphoreType.DMA((6,)),
        ],
        compiler_params=pltpu.CompilerParams(
            collective_id=0,
            vmem_limit_bytes=100 * 1024 * 1024,
        ),
    )(Qp, Kp, Vp)

    return out_p.transpose(0, 2, 1, 3).reshape(b, s_loc, h, d)


# device time: 58050 ns/iter; 3.6130x vs baseline; 1.1168x over previous
import functools
import os

import jax
import jax.numpy as jnp
from jax import lax
from jax.experimental import pallas as pl
from jax.experimental.pallas import tpu as pltpu

N_DEV = 4

_SKIP_RING = bool(int(os.environ.get("KERNEL_SKIP_RING", "0")))
_SKIP_COMPUTE = bool(int(os.environ.get("KERNEL_SKIP_COMPUTE", "0")))
_RING_MODE = os.environ.get("KERNEL_RING_MODE", "full")
_COMPUTE_MODE = int(os.environ.get("KERNEL_COMPUTE_MODE", "0"))


def kernel(Q, K, V):
    b, s_loc, h, d = Q.shape
    h2 = h // 2
    d2 = 2 * d
    scale = d ** -0.5

    def prep(x):
        return x.reshape(b, s_loc, h2, d2).transpose(0, 2, 1, 3)

    Qp = prep((Q * scale).astype(jnp.bfloat16))
    Kp = prep(K.astype(jnp.bfloat16))
    Vp = prep(V.astype(jnp.bfloat16))

    def body(q_ref, k_ref, v_ref, out_ref,
             kbuf, vbuf, accbuf, lbuf, send_sems, recv_sems):
        my = lax.axis_index("i")
        left = (my - 1) % N_DEV
        right = (my + 1) % N_DEV

        barrier = pltpu.get_barrier_semaphore()
        for nbr in (left, right):
            pl.semaphore_signal(
                barrier, inc=1,
                device_id=(nbr,), device_id_type=pl.DeviceIdType.MESH,
            )
        pl.semaphore_wait(barrier, 2)

        def remote(src, dst, sem_idx, target):
            return pltpu.make_async_remote_copy(
                src_ref=src, dst_ref=dst,
                send_sem=send_sems.at[sem_idx],
                recv_sem=recv_sems.at[sem_idx],
                device_id=(target,),
                device_id_type=pl.DeviceIdType.MESH,
            )

        H = [slice(0, h2 // 2), slice(h2 // 2, h2)]
        k_r = [remote(k_ref.at[:, H[i]], kbuf.at[0, :, H[i]], 0 + i, right)
               for i in range(2)]
        v_r = [remote(v_ref.at[:, H[i]], vbuf.at[0, :, H[i]], 2 + i, right)
               for i in range(2)]
        k_l = [remote(k_ref.at[:, H[i]], kbuf.at[1, :, H[i]], 4 + i, left)
               for i in range(2)]
        v_l = [remote(v_ref.at[:, H[i]], vbuf.at[1, :, H[i]], 6 + i, left)
               for i in range(2)]
        k_fwd = [remote(kbuf.at[0, :, H[i]], kbuf.at[2, :, H[i]], 8 + i, right)
                 for i in range(2)]
        v_fwd = [remote(vbuf.at[1, :, H[i]], vbuf.at[2, :, H[i]], 10 + i, left)
                 for i in range(2)]

        _do_v = _RING_MODE in ("full", "hop0")
        _do_fwd = _RING_MODE == "full"
        if not _SKIP_RING:
            for i in range(2):
                k_r[i].start()
                if _do_v:
                    v_r[i].start()
            for i in range(2):
                k_l[i].start()
                if _do_v:
                    v_l[i].start()

        def head_chunk(q_t, k_t, v_t):
            st = lax.dot_general(
                q_t, k_t,
                (((1,), (1,)), ((), ())),
                preferred_element_type=jnp.float32,
            )
            p = st if _COMPUTE_MODE in (1, 3) else jnp.exp(st)
            if _COMPUTE_MODE in (2, 3):
                lsum = jnp.ones((s_loc, 1), jnp.float32)
            else:
                lsum = p.sum(axis=1, keepdims=True)
            pv = lax.dot_general(
                p.astype(jnp.bfloat16), v_t,
                (((1,), (0,)), ((), ())),
                preferred_element_type=jnp.float32,
            )
            return pv, lsum

        def make_phase(slots, mode, half=None):
            lo = 0 if half is None else half * (h2 // 2)
            n = h2 if half is None else h2 // 2

            def step(idx, c):
                bb = idx // n
                hp = lo + idx % n
                qq = q_ref[bb, hp]
                pvs = [jnp.zeros((s_loc, d), jnp.float32) for _ in range(2)]
                lsums = [jnp.zeros((s_loc, 1), jnp.float32) for _ in range(2)]
                for j in slots:
                    kk2 = (k_ref if j is None else kbuf.at[j])[bb, hp]
                    vv2 = (v_ref if j is None else vbuf.at[j])[bb, hp]
                    for t in range(2):
                        pv, ls = head_chunk(
                            qq[:, t * d:(t + 1) * d],
                            kk2[:, t * d:(t + 1) * d],
                            vv2[:, t * d:(t + 1) * d],
                        )
                        pvs[t] = pvs[t] + pv
                        lsums[t] = lsums[t] + ls
                pv = jnp.concatenate(pvs, axis=1)
                if mode == "first":
                    accbuf[bb, hp] = pv
                    for t in range(2):
                        lbuf[bb, 2 * hp + t] = lsums[t]
                elif mode == "mid":
                    accbuf[bb, hp] = accbuf[bb, hp] + pv
                    for t in range(2):
                        lbuf[bb, 2 * hp + t] = lbuf[bb, 2 * hp + t] + lsums[t]
                else:
                    tot = accbuf[bb, hp] + pv
                    outs = [
                        tot[:, t * d:(t + 1) * d]
                        / (lbuf[bb, 2 * hp + t] + lsums[t])
                        for t in range(2)
                    ]
                    out_ref[bb, hp] = jnp.concatenate(
                        outs, axis=1).astype(jnp.bfloat16)
                return c

            if not _SKIP_COMPUTE:
                lax.fori_loop(0, b * n, step, 0)

        make_phase([None], "first")

        for i in range(2):
            if not _SKIP_RING:
                k_r[i].wait_recv()
                if _do_fwd:
                    k_fwd[i].start()
                k_l[i].wait_recv()
                if _do_v:
                    v_l[i].wait_recv()
                    if _do_fwd:
                        v_fwd[i].start()
                    v_r[i].wait_recv()
            make_phase([0, 1], "mid", half=i)

        for i in range(2):
            if not _SKIP_RING and _do_fwd:
                k_fwd[i].wait_recv()
                v_fwd[i].wait_recv()
            make_phase([2], "last", half=i)

        if not _SKIP_RING:
            drain = k_r + k_l
            if _do_v:
                drain += v_r + v_l
            if _do_fwd:
                drain += k_fwd + v_fwd
            for r in drain:
                r.wait_send()

        @functools.partial(
            pl.run_scoped, second_barrier=pltpu.SemaphoreType.REGULAR
        )
        def _(second_barrier):
            for nbr in (left, right):
                pl.semaphore_signal(
                    second_barrier, inc=1,
                    device_id=(nbr,), device_id_type=pl.DeviceIdType.MESH,
                )
            pl.semaphore_wait(second_barrier, 2)

    out_p = pl.pallas_call(
        body,
        out_shape=jax.ShapeDtypeStruct((b, h2, s_loc, d2), jnp.bfloat16),
        in_specs=[
            pl.BlockSpec(memory_space=pltpu.VMEM),
            pl.BlockSpec(memory_space=pltpu.VMEM),
            pl.BlockSpec(memory_space=pltpu.VMEM),
        ],
        out_specs=pl.BlockSpec(memory_space=pltpu.VMEM),
        scratch_shapes=[
            pltpu.VMEM((3, b, h2, s_loc, d2), jnp.bfloat16),
            pltpu.VMEM((3, b, h2, s_loc, d2), jnp.bfloat16),
            pltpu.VMEM((b, h2, s_loc, d2), jnp.float32),
            pltpu.VMEM((b, h, s_loc, 1), jnp.float32),
            pltpu.SemaphoreType.DMA((12,)),
            pltpu.SemaphoreType.DMA((12,)),
        ],
        compiler_params=pltpu.CompilerParams(
            collective_id=0,
            vmem_limit_bytes=100 * 1024 * 1024,
        ),
    )(Qp, Kp, Vp)

    return out_p.transpose(0, 2, 1, 3).reshape(b, s_loc, h, d)
